# baseline (device time: 95607 ns/iter reference)
import jax
import jax.numpy as jnp
from jax import lax
from jax.experimental import pallas as pl
from jax.experimental.pallas import tpu as pltpu

N_DEV = 8
B, SQ, D_MODEL = 2, 256, 512
SKV_LOC = 256
HQ = 32
HQ_LOC = 4
DH = 64
BLK = 64
NC = 2 * B
ROWS = SQ // N_DEV


def kernel(x, Wq, K_ext, V_ext, Wo):
    def body(x_ref, wq_ref, k_ref, v_ref, wo_ref, out_ref,
             ksrc, vsrc, kbuf, vbuf, psrc, rbuf, gbuf, fbuf, ctx_ref,
             ksend, krecv, vsend, vrecv, rsend, rrecv, asend, arecv):
        me = lax.axis_index("i")

        ksrc[...] = k_ref[...].astype(jnp.float8_e4m3fn).transpose(2, 0, 1, 3)
        vsrc[...] = v_ref[...].astype(jnp.bfloat16).transpose(2, 0, 1, 3)

        kv_rdmas = []
        for k in range(1, N_DEV):
            t = (me + k) % N_DEV
            for src, buf, ssem, rsem in (
                (ksrc, kbuf, ksend, krecv),
                (vsrc, vbuf, vsend, vrecv),
            ):
                r = pltpu.make_async_remote_copy(
                    src_ref=src.at[pl.ds(t * HQ_LOC, HQ_LOC)],
                    dst_ref=buf.at[k],
                    send_sem=ssem.at[k],
                    recv_sem=rsem.at[k],
                    device_id=(t,),
                    device_id_type=pl.DeviceIdType.MESH,
                )
                r.start()
                kv_rdmas.append(r)

        kbuf[0] = ksrc[pl.ds(me * HQ_LOC, HQ_LOC)]
        vbuf[0] = vsrc[pl.ds(me * HQ_LOC, HQ_LOC)]
        q = jnp.dot(
            x_ref[...].reshape(B * SQ, D_MODEL), wq_ref[...],
            preferred_element_type=jnp.float32,
        ).reshape(B, SQ, HQ_LOC, DH).astype(jnp.bfloat16)

        for r in kv_rdmas:
            r.wait()

        kg = kbuf[...].astype(jnp.bfloat16)
        vg = vbuf[...]
        nsel = N_DEV * BLK
        for b in range(B):
            for h in range(HQ_LOC):
                for qb in range(SQ // BLK):
                    lo = qb * BLK
                    qh = q[b, lo:lo + BLK, h, :]
                    kh = kg[:, h, b, lo:lo + BLK].reshape(nsel, DH)
                    vh = vg[:, h, b, lo:lo + BLK].reshape(nsel, DH)
                    s = lax.dot_general(
                        qh, kh, (((1,), (1,)), ((), ())),
                        preferred_element_type=jnp.float32,
                    ) * 0.125
                    w = jnp.exp(s)
                    l = jnp.sum(w, axis=-1, keepdims=True)
                    c = jnp.dot(
                        w.astype(jnp.bfloat16), vh,
                        preferred_element_type=jnp.float32,
                    )
                    ctx_ref[b, lo:lo + BLK, h, :] = c / l

        partial = jnp.dot(
            ctx_ref[...].reshape(B * SQ, HQ_LOC * DH), wo_ref[...],
            preferred_element_type=jnp.float32,
        ).reshape(B, SQ, D_MODEL)
        psrc[...] = partial.astype(jnp.bfloat16)
        rbuf[0] = psrc[:, pl.ds(me * ROWS, ROWS), :]

        rs_rdmas = []
        for k in range(1, N_DEV):
            t = (me + k) % N_DEV
            r = pltpu.make_async_remote_copy(
                src_ref=psrc.at[:, pl.ds(t * ROWS, ROWS), :],
                dst_ref=rbuf.at[k],
                send_sem=rsend.at[k],
                recv_sem=rrecv.at[k],
                device_id=(t,),
                device_id_type=pl.DeviceIdType.MESH,
            )
            r.start()
            rs_rdmas.append(r)
        for r in rs_rdmas:
            r.wait()

        mine = jnp.sum(rbuf[...].astype(jnp.float32), axis=0)
        out_ref[:, pl.ds(me * ROWS, ROWS), :] = mine
        gbuf[...] = mine.astype(jnp.bfloat16)

        ag_rdmas = []
        for k in range(1, N_DEV):
            t = (me + k) % N_DEV
            r = pltpu.make_async_remote_copy(
                src_ref=gbuf,
                dst_ref=fbuf.at[k],
                send_sem=asend.at[k],
                recv_sem=arecv.at[k],
                device_id=(t,),
                device_id_type=pl.DeviceIdType.MESH,
            )
            r.start()
            ag_rdmas.append(r)
        for r in ag_rdmas:
            r.wait()
        for k in range(1, N_DEV):
            src = (me - k) % N_DEV
            out_ref[:, pl.ds(src * ROWS, ROWS), :] = fbuf[k].astype(jnp.float32)

    bf = jnp.bfloat16
    return pl.pallas_call(
        body,
        out_shape=jax.ShapeDtypeStruct((B, SQ, D_MODEL), jnp.float32),
        in_specs=[pl.BlockSpec(memory_space=pltpu.VMEM)] * 5,
        out_specs=pl.BlockSpec(memory_space=pltpu.VMEM),
        scratch_shapes=[
            pltpu.VMEM((HQ, B, SKV_LOC, DH), jnp.float8_e4m3fn),
            pltpu.VMEM((HQ, B, SKV_LOC, DH), bf),
            pltpu.VMEM((N_DEV, HQ_LOC, B, SKV_LOC, DH), jnp.float8_e4m3fn),
            pltpu.VMEM((N_DEV, HQ_LOC, B, SKV_LOC, DH), bf),
            pltpu.VMEM((B, SQ, D_MODEL), bf),
            pltpu.VMEM((N_DEV, B, ROWS, D_MODEL), bf),
            pltpu.VMEM((B, ROWS, D_MODEL), bf),
            pltpu.VMEM((N_DEV, B, ROWS, D_MODEL), bf),
            pltpu.VMEM((B, SQ, HQ_LOC, DH), jnp.float32),
            pltpu.SemaphoreType.DMA((N_DEV,)),
            pltpu.SemaphoreType.DMA((N_DEV,)),
            pltpu.SemaphoreType.DMA((N_DEV,)),
            pltpu.SemaphoreType.DMA((N_DEV,)),
            pltpu.SemaphoreType.DMA((N_DEV,)),
            pltpu.SemaphoreType.DMA((N_DEV,)),
            pltpu.SemaphoreType.DMA((N_DEV,)),
            pltpu.SemaphoreType.DMA((N_DEV,)),
        ],
    )(x, Wq, K_ext, V_ext, Wo)


# device time: 88181 ns/iter; 1.0842x vs baseline; 1.0842x over previous
import jax
import jax.numpy as jnp
from jax import lax
from jax.experimental import pallas as pl
from jax.experimental.pallas import tpu as pltpu

N_DEV = 8
B, SQ, D_MODEL = 2, 256, 512
SKV_LOC = 256
HQ = 32
HQ_LOC = 4
DH = 64
BLK = 64
NC = 2 * B
ROWS = SQ // N_DEV


def kernel(x, Wq, K_ext, V_ext, Wo):
    def body(x_ref, wq_ref, k_ref, v_ref, wo_ref, out_ref,
             ksrc, vsrc, kbuf, vbuf, psrc, rbuf, gbuf, fbuf, ctx_ref,
             ksend, krecv, vsend, vrecv, rsend, rrecv, asend, arecv):
        me = lax.axis_index("i")

        kv_rdmas = []
        for src_ref, src, buf, ssem, rsem, dt in (
            (k_ref, ksrc, kbuf, ksend, krecv, jnp.float8_e4m3fn),
            (v_ref, vsrc, vbuf, vsend, vrecv, jnp.bfloat16),
        ):
            for k in range(1, N_DEV):
                t = (me + k) % N_DEV
                sl = pl.ds(t * HQ_LOC, HQ_LOC)
                src[sl] = (
                    src_ref[:, :, sl, :].astype(dt).transpose(2, 0, 1, 3)
                )
                r = pltpu.make_async_remote_copy(
                    src_ref=src.at[sl],
                    dst_ref=buf.at[k],
                    send_sem=ssem.at[k],
                    recv_sem=rsem.at[k],
                    device_id=(t,),
                    device_id_type=pl.DeviceIdType.MESH,
                )
                r.start()
                kv_rdmas.append(r)

        own = pl.ds(me * HQ_LOC, HQ_LOC)
        kbuf[0] = k_ref[:, :, own, :].astype(jnp.float8_e4m3fn).transpose(2, 0, 1, 3)
        vbuf[0] = v_ref[:, :, own, :].astype(jnp.bfloat16).transpose(2, 0, 1, 3)
        q = jnp.dot(
            x_ref[...].reshape(B * SQ, D_MODEL), wq_ref[...],
            preferred_element_type=jnp.float32,
        ).reshape(B, SQ, HQ_LOC, DH).astype(jnp.bfloat16)

        for r in kv_rdmas:
            r.wait()

        kg = kbuf[...].astype(jnp.bfloat16)
        vg = vbuf[...]
        nsel = N_DEV * BLK
        wo = wo_ref[...]
        for qb in range(SQ // BLK):
            lo = qb * BLK
            for b in range(B):
                for h in range(HQ_LOC):
                    qh = q[b, lo:lo + BLK, h, :]
                    kh = kg[:, h, b, lo:lo + BLK].reshape(nsel, DH)
                    vh = vg[:, h, b, lo:lo + BLK].reshape(nsel, DH)
                    s = lax.dot_general(
                        qh, kh, (((1,), (1,)), ((), ())),
                        preferred_element_type=jnp.float32,
                    ) * 0.125
                    w = jnp.exp(s)
                    l = jnp.sum(w, axis=-1, keepdims=True)
                    c = jnp.dot(
                        w.astype(jnp.bfloat16), vh,
                        preferred_element_type=jnp.float32,
                    )
                    ctx_ref[b, lo:lo + BLK, h, :] = c / l
            blk = ctx_ref[:, lo:lo + BLK].reshape(B * BLK, HQ_LOC * DH)
            pr = jnp.dot(blk, wo, preferred_element_type=jnp.float32)
            psrc[:, lo:lo + BLK, :] = (
                pr.reshape(B, BLK, D_MODEL).astype(jnp.bfloat16)
            )
            for k in range(1, N_DEV):
                t = (me + k) % N_DEV

                @pl.when(t // (BLK // ROWS) == qb)
                def _(k=k, t=t):
                    pltpu.make_async_remote_copy(
                        src_ref=psrc.at[:, pl.ds(t * ROWS, ROWS), :],
                        dst_ref=rbuf.at[k],
                        send_sem=rsend.at[k],
                        recv_sem=rrecv.at[k],
                        device_id=(t,),
                        device_id_type=pl.DeviceIdType.MESH,
                    ).start()

        rbuf[0] = psrc[:, pl.ds(me * ROWS, ROWS), :]
        for k in range(1, N_DEV):
            t = (me + k) % N_DEV
            pltpu.make_async_remote_copy(
                src_ref=psrc.at[:, pl.ds(t * ROWS, ROWS), :],
                dst_ref=rbuf.at[k],
                send_sem=rsend.at[k],
                recv_sem=rrecv.at[k],
                device_id=(t,),
                device_id_type=pl.DeviceIdType.MESH,
            ).wait()

        mine = jnp.sum(rbuf[...].astype(jnp.float32), axis=0)
        out_ref[:, pl.ds(me * ROWS, ROWS), :] = mine
        gbuf[...] = mine.astype(jnp.bfloat16)

        ag_rdmas = []
        for k in range(1, N_DEV):
            t = (me + k) % N_DEV
            r = pltpu.make_async_remote_copy(
                src_ref=gbuf,
                dst_ref=fbuf.at[k],
                send_sem=asend.at[k],
                recv_sem=arecv.at[k],
                device_id=(t,),
                device_id_type=pl.DeviceIdType.MESH,
            )
            r.start()
            ag_rdmas.append(r)
        for r in ag_rdmas:
            r.wait()
        for k in range(1, N_DEV):
            src = (me - k) % N_DEV
            out_ref[:, pl.ds(src * ROWS, ROWS), :] = fbuf[k].astype(jnp.float32)

    bf = jnp.bfloat16
    return pl.pallas_call(
        body,
        out_shape=jax.ShapeDtypeStruct((B, SQ, D_MODEL), jnp.float32),
        in_specs=[pl.BlockSpec(memory_space=pltpu.VMEM)] * 5,
        out_specs=pl.BlockSpec(memory_space=pltpu.VMEM),
        scratch_shapes=[
            pltpu.VMEM((HQ, B, SKV_LOC, DH), jnp.float8_e4m3fn),
            pltpu.VMEM((HQ, B, SKV_LOC, DH), bf),
            pltpu.VMEM((N_DEV, HQ_LOC, B, SKV_LOC, DH), jnp.float8_e4m3fn),
            pltpu.VMEM((N_DEV, HQ_LOC, B, SKV_LOC, DH), bf),
            pltpu.VMEM((B, SQ, D_MODEL), bf),
            pltpu.VMEM((N_DEV, B, ROWS, D_MODEL), bf),
            pltpu.VMEM((B, ROWS, D_MODEL), bf),
            pltpu.VMEM((N_DEV, B, ROWS, D_MODEL), bf),
            pltpu.VMEM((B, SQ, HQ_LOC, DH), jnp.float32),
            pltpu.SemaphoreType.DMA((N_DEV,)),
            pltpu.SemaphoreType.DMA((N_DEV,)),
            pltpu.SemaphoreType.DMA((N_DEV,)),
            pltpu.SemaphoreType.DMA((N_DEV,)),
            pltpu.SemaphoreType.DMA((N_DEV,)),
            pltpu.SemaphoreType.DMA((N_DEV,)),
            pltpu.SemaphoreType.DMA((N_DEV,)),
            pltpu.SemaphoreType.DMA((N_DEV,)),
        ],
    )(x, Wq, K_ext, V_ext, Wo)


# device time: 83262 ns/iter; 1.1483x vs baseline; 1.0591x over previous
import jax
import jax.numpy as jnp
from jax import lax
from jax.experimental import pallas as pl
from jax.experimental.pallas import tpu as pltpu

N_DEV = 8
B, SQ, D_MODEL = 2, 256, 512
SKV_LOC = 256
HQ = 32
HQ_LOC = 4
DH = 64
BLK = 64
NC = 2 * B
ROWS = SQ // N_DEV


def kernel(x, Wq, K_ext, V_ext, Wo):
    def body(x_ref, wq_ref, k_ref, v_ref, wo_ref, out_ref,
             ksrc, vsrc, kbuf, vbuf, psrc, rbuf, gbuf, fbuf, ctx_ref, wbuf,
             ksend, krecv, vsend, vrecv, rsend, rrecv, asend, arecv):
        me = lax.axis_index("i")

        k_rdmas = []
        v_rdmas = []
        for src_ref, src, buf, ssem, rsem, dt, rdmas in (
            (k_ref, ksrc, kbuf, ksend, krecv, jnp.float8_e4m3fn, k_rdmas),
            (v_ref, vsrc, vbuf, vsend, vrecv, jnp.bfloat16, v_rdmas),
        ):
            for k in range(1, N_DEV):
                t = (me + k) % N_DEV
                sl = pl.ds(t * HQ_LOC, HQ_LOC)
                src[sl] = (
                    src_ref[:, :, sl, :].astype(dt).transpose(2, 0, 1, 3)
                )
                r = pltpu.make_async_remote_copy(
                    src_ref=src.at[sl],
                    dst_ref=buf.at[k],
                    send_sem=ssem.at[k],
                    recv_sem=rsem.at[k],
                    device_id=(t,),
                    device_id_type=pl.DeviceIdType.MESH,
                )
                r.start()
                rdmas.append(r)

        own = pl.ds(me * HQ_LOC, HQ_LOC)
        kbuf[0] = k_ref[:, :, own, :].astype(jnp.float8_e4m3fn).transpose(2, 0, 1, 3)
        vbuf[0] = v_ref[:, :, own, :].astype(jnp.bfloat16).transpose(2, 0, 1, 3)
        q = jnp.dot(
            x_ref[...].reshape(B * SQ, D_MODEL), wq_ref[...],
            preferred_element_type=jnp.float32,
        ).reshape(B, SQ, HQ_LOC, DH).astype(jnp.bfloat16)

        for r in k_rdmas:
            r.wait()
        kg = kbuf[...].astype(jnp.bfloat16)
        nsel = N_DEV * BLK
        for b in range(B):
            for h in range(HQ_LOC):
                for qb in range(SQ // BLK):
                    lo = qb * BLK
                    qh = q[b, lo:lo + BLK, h, :]
                    kh = kg[:, h, b, lo:lo + BLK].reshape(nsel, DH)
                    s = lax.dot_general(
                        qh, kh, (((1,), (1,)), ((), ())),
                        preferred_element_type=jnp.float32,
                    ) * 0.125
                    w = jnp.exp(s)
                    l = jnp.sum(w, axis=-1, keepdims=True)
                    wbuf[b, h, qb] = (w / l).astype(jnp.bfloat16)

        for r in v_rdmas:
            r.wait()
        vg = vbuf[...]
        wo = wo_ref[...]
        for qb in range(SQ // BLK):
            lo = qb * BLK
            for b in range(B):
                for h in range(HQ_LOC):
                    vh = vg[:, h, b, lo:lo + BLK].reshape(nsel, DH)
                    ctx_ref[b, lo:lo + BLK, h, :] = jnp.dot(
                        wbuf[b, h, qb], vh,
                        preferred_element_type=jnp.float32,
                    )
            blk = ctx_ref[:, lo:lo + BLK].reshape(B * BLK, HQ_LOC * DH)
            pr = jnp.dot(blk, wo, preferred_element_type=jnp.float32)
            psrc[:, lo:lo + BLK, :] = (
                pr.reshape(B, BLK, D_MODEL).astype(jnp.bfloat16)
            )
            for k in range(1, N_DEV):
                t = (me + k) % N_DEV

                @pl.when(t // (BLK // ROWS) == qb)
                def _(k=k, t=t):
                    pltpu.make_async_remote_copy(
                        src_ref=psrc.at[:, pl.ds(t * ROWS, ROWS), :],
                        dst_ref=rbuf.at[k],
                        send_sem=rsend.at[k],
                        recv_sem=rrecv.at[k],
                        device_id=(t,),
                        device_id_type=pl.DeviceIdType.MESH,
                    ).start()

        rbuf[0] = psrc[:, pl.ds(me * ROWS, ROWS), :]
        for k in range(1, N_DEV):
            t = (me + k) % N_DEV
            pltpu.make_async_remote_copy(
                src_ref=psrc.at[:, pl.ds(t * ROWS, ROWS), :],
                dst_ref=rbuf.at[k],
                send_sem=rsend.at[k],
                recv_sem=rrecv.at[k],
                device_id=(t,),
                device_id_type=pl.DeviceIdType.MESH,
            ).wait()

        mine = jnp.sum(rbuf[...].astype(jnp.float32), axis=0)
        out_ref[:, pl.ds(me * ROWS, ROWS), :] = mine
        gbuf[...] = mine.astype(jnp.bfloat16)

        ag_rdmas = []
        for k in range(1, N_DEV):
            t = (me + k) % N_DEV
            r = pltpu.make_async_remote_copy(
                src_ref=gbuf,
                dst_ref=fbuf.at[k],
                send_sem=asend.at[k],
                recv_sem=arecv.at[k],
                device_id=(t,),
                device_id_type=pl.DeviceIdType.MESH,
            )
            r.start()
            ag_rdmas.append(r)
        for r in ag_rdmas:
            r.wait()
        for k in range(1, N_DEV):
            src = (me - k) % N_DEV
            out_ref[:, pl.ds(src * ROWS, ROWS), :] = fbuf[k].astype(jnp.float32)

    bf = jnp.bfloat16
    return pl.pallas_call(
        body,
        out_shape=jax.ShapeDtypeStruct((B, SQ, D_MODEL), jnp.float32),
        in_specs=[pl.BlockSpec(memory_space=pltpu.VMEM)] * 5,
        out_specs=pl.BlockSpec(memory_space=pltpu.VMEM),
        scratch_shapes=[
            pltpu.VMEM((HQ, B, SKV_LOC, DH), jnp.float8_e4m3fn),
            pltpu.VMEM((HQ, B, SKV_LOC, DH), bf),
            pltpu.VMEM((N_DEV, HQ_LOC, B, SKV_LOC, DH), jnp.float8_e4m3fn),
            pltpu.VMEM((N_DEV, HQ_LOC, B, SKV_LOC, DH), bf),
            pltpu.VMEM((B, SQ, D_MODEL), bf),
            pltpu.VMEM((N_DEV, B, ROWS, D_MODEL), bf),
            pltpu.VMEM((B, ROWS, D_MODEL), bf),
            pltpu.VMEM((N_DEV, B, ROWS, D_MODEL), bf),
            pltpu.VMEM((B, SQ, HQ_LOC, DH), jnp.float32),
            pltpu.VMEM((B, HQ_LOC, SQ // BLK, BLK, N_DEV * BLK), bf),
            pltpu.SemaphoreType.DMA((N_DEV,)),
            pltpu.SemaphoreType.DMA((N_DEV,)),
            pltpu.SemaphoreType.DMA((N_DEV,)),
            pltpu.SemaphoreType.DMA((N_DEV,)),
            pltpu.SemaphoreType.DMA((N_DEV,)),
            pltpu.SemaphoreType.DMA((N_DEV,)),
            pltpu.SemaphoreType.DMA((N_DEV,)),
            pltpu.SemaphoreType.DMA((N_DEV,)),
        ],
    )(x, Wq, K_ext, V_ext, Wo)
